# Initial kernel scaffold; baseline (speedup 1.0000x reference)
import jax
import jax.numpy as jnp
from jax.experimental import pallas as pl


def kernel(vert, vol):
    raise NotImplementedError("write your pallas kernel here")


# speedup vs baseline: 3.2652x; 3.2652x over previous
import functools

import jax
import jax.numpy as jnp
import numpy as np
from jax import lax
from jax.experimental import pallas as pl
from jax.experimental.pallas import tpu as pltpu
from jax.experimental.pallas import tpu_sc as plsc

_D1, _D2, _D3 = 112, 224, 160
_C = 8
_R = _D1 * _D2 * _D3
_V = 150000

_NC, _NS = 2, 16
_NW = _NC * _NS
_BC = 480
_NCHUNK = 10
_T = _BC * _NCHUNK
_V_PAD = _NW * _T
_IDX_ROWS = _BC * 8 // 128
_P = _BC // 2

_SX = _D2 * _D3
_SY = _D3

_XMAX = np.float32(np.float32(_D1) - np.float32(1.0) - np.float32(1e-5))
_YMAX = np.float32(np.float32(_D2) - np.float32(1.0) - np.float32(1e-5))
_ZMAX = np.float32(np.float32(_D3) - np.float32(1.0) - np.float32(1e-5))
_EPS = np.float32(1e-5)

_mesh = plsc.VectorSubcoreMesh(core_axis_name="c", subcore_axis_name="s")


@functools.partial(
    pl.kernel,
    mesh=_mesh,
    out_type=jax.ShapeDtypeStruct((_V_PAD // 2, 16), jnp.float32),
    scratch_types=[
        pltpu.VMEM((_BC,), jnp.float32),
        pltpu.VMEM((_BC,), jnp.float32),
        pltpu.VMEM((_BC,), jnp.float32),
        pltpu.VMEM((_BC * 8,), jnp.int32),
        pltpu.VMEM((4, _BC), jnp.float32),
        pltpu.VMEM((_BC,), jnp.float32),
        pltpu.VMEM((_BC * 8, 8), jnp.float32),
        pltpu.VMEM((_P, 16), jnp.float32),
        pltpu.SemaphoreType.DMA,
    ],
    compiler_params=pltpu.CompilerParams(needs_layout_passes=False,
                                         use_tc_tiling_on_sc=False),
)
def _interp(table, xs, ys, zs, out, x_v, y_v, z_v, idx_v, w_v, uz_v, g_v,
            o_v, sem):
    wid = lax.axis_index("s") * _NC + lax.axis_index("c")

    iota = lax.iota(jnp.int32, 16)
    pat01 = iota >> 3
    patc = iota & 7
    kconst = [jnp.full((16,), k, jnp.int32) for k in range(4)]

    def chunk_body(c, carry):
        base = wid * _T + c * _BC
        pltpu.sync_copy(xs.at[pl.ds(base, _BC)], x_v)
        pltpu.sync_copy(ys.at[pl.ds(base, _BC)], y_v)
        pltpu.sync_copy(zs.at[pl.ds(base, _BC)], z_v)

        def idx_body(i, carry2):
            off16 = i * 16
            xv = x_v[pl.ds(off16, 16)]
            yv = y_v[pl.ds(off16, 16)]
            zv = z_v[pl.ds(off16, 16)]
            xc = jnp.minimum(jnp.maximum(xv, _EPS), _XMAX)
            yc = jnp.minimum(jnp.maximum(yv, _EPS), _YMAX)
            zc = jnp.minimum(jnp.maximum(zv, _EPS), _ZMAX)
            xi = xc.astype(jnp.int32)
            yi = yc.astype(jnp.int32)
            zi = zc.astype(jnp.int32)
            ux = xc - xi.astype(jnp.float32)
            uy = yc - yi.astype(jnp.float32)
            uz = zc - zi.astype(jnp.float32)
            rbase = xi * 35840 + yi * 16
            za = ((zi >> 4) * 3584 + (zi & 15)) + rbase
            z1 = zi + 1
            zb = ((z1 >> 4) * 3584 + (z1 & 15)) + rbase

            for k, (dx, dy) in enumerate(((0, 0), (0, 1), (1, 0), (1, 1))):
                roff = dx * 35840 + dy * 16
                for dz in (0, 1):
                    s = k * 2 + dz
                    idx_v[pl.ds(i * 128 + s * 16, 16)] = (zb if dz else za) + roff

            mx = np.float32(1.0) - ux
            my = np.float32(1.0) - uy
            w_v[0, pl.ds(off16, 16)] = mx * my
            w_v[1, pl.ds(off16, 16)] = mx * uy
            w_v[2, pl.ds(off16, 16)] = ux * my
            w_v[3, pl.ds(off16, 16)] = ux * uy
            uz_v[pl.ds(off16, 16)] = uz
            return carry2

        lax.fori_loop(0, _BC // 16, idx_body, 0)

        for j in range(_IDX_ROWS):
            pltpu.async_copy(table.at[idx_v.at[pl.ds(j * 128, 128)]],
                             g_v.at[pl.ds(j * 128, 128)], sem)
        for j in range(_IDX_ROWS):
            pltpu.make_async_copy(table.at[idx_v.at[pl.ds(j * 128, 128)]],
                                  g_v.at[pl.ds(j * 128, 128)], sem).wait()

        def pair_body(p, carry2):
            rowbase = ((p >> 3) << 7) + ((p & 7) << 1)
            col2p = 2 * p + pat01
            acc_a = None
            acc_b = None
            for k in range(4):
                row_a = rowbase + k * 32 + pat01
                a = plsc.load_gather(g_v, [row_a, patc])
                b = plsc.load_gather(g_v, [row_a + 16, patc])
                wk = plsc.load_gather(w_v, [kconst[k], col2p])
                if k == 0:
                    acc_a = wk * a
                    acc_b = wk * b
                else:
                    acc_a = acc_a + wk * a
                    acc_b = acc_b + wk * b
            uzp = plsc.load_gather(uz_v, [col2p])
            o_v[p, :] = acc_a + uzp * (acc_b - acc_a)
            return carry2

        lax.fori_loop(0, _P, pair_body, 0)

        pltpu.sync_copy(o_v, out.at[pl.ds(wid * (_T // 2) + c * _P, _P)])
        return carry

    lax.fori_loop(0, _NCHUNK, chunk_body, 0)


_ZG = _D3 // 16
_UNITS = _D1 * _ZG
_UPW = _UNITS // _NW


@functools.partial(
    pl.kernel,
    mesh=_mesh,
    out_type=jax.ShapeDtypeStruct((_R * _C // 128, 128), jnp.float32),
    scratch_types=[
        pltpu.VMEM((2, _C, 16, _D2), jnp.float32),
        pltpu.VMEM((2, _D2, 128), jnp.float32),
        pltpu.SemaphoreType.DMA,
        pltpu.SemaphoreType.DMA,
    ],
    compiler_params=pltpu.CompilerParams(needs_layout_passes=False,
                                         use_tc_tiling_on_sc=True),
)
def _build_table(vol5, out, buf2, obuf2, semi, semo):
    wid = lax.axis_index("s") * _NC + lax.axis_index("c")
    u0 = wid * _UPW

    iota = lax.iota(jnp.int32, 16)
    pat01 = iota >> 3
    patc = iota & 7
    zpats = [2 * m + pat01 for m in range(8)]

    def unit_xzg(unit):
        return unit // _ZG, unit % _ZG

    def fire_inputs(unit, slot):
        x, zg = unit_xzg(unit)
        for c in range(_C):
            pltpu.async_copy(vol5.at[0, c, x, pl.ds(zg * 16, 16), :],
                             buf2.at[slot, c], semi)

    def wait_inputs(unit, slot):
        x, zg = unit_xzg(unit)
        for c in range(_C):
            pltpu.make_async_copy(vol5.at[0, c, x, pl.ds(zg * 16, 16), :],
                                  buf2.at[slot, c], semi).wait()

    def out_slice(unit):
        x, zg = unit_xzg(unit)
        return out.at[pl.ds((x * _ZG + zg) * _D2, _D2)]

    fire_inputs(u0, 0)

    def unit_body(i, carry):
        unit = u0 + i
        slot = i & 1
        wait_inputs(unit, slot)

        @pl.when(i + 1 < _UPW)
        def _():
            fire_inputs(unit + 1, (i + 1) & 1)

        @pl.when(i >= 2)
        def _():
            pltpu.make_async_copy(obuf2.at[slot], out_slice(unit), semo).wait()

        slotv = jnp.full((16,), slot, jnp.int32)

        def y_body(y, carry2):
            yv = jnp.full((16,), y, jnp.int32)
            vals = [plsc.load_gather(buf2, [slotv, patc, zpats[m], yv])
                    for m in range(8)]
            for m in range(8):
                obuf2[slot, y, pl.ds(m * 16, 16)] = vals[m]
            return carry2

        lax.fori_loop(0, _D2, y_body, 0)

        pltpu.async_copy(obuf2.at[slot], out_slice(unit), semo)
        return carry

    lax.fori_loop(0, _UPW, unit_body, 0)

    for k in (2, 1):
        pltpu.make_async_copy(obuf2.at[0], out_slice(u0 + _UPW - k),
                              semo).wait()


def kernel(vert, vol):
    verts = vert[0]
    table = _build_table(jnp.swapaxes(vol, 3, 4)).reshape(_R, _C)
    vpad = jnp.pad(verts, ((0, _V_PAD - _V), (0, 0)))
    xs = vpad[:, 0]
    ys = vpad[:, 1]
    zs = vpad[:, 2]
    out2 = _interp(table, xs, ys, zs)
    return out2.reshape(_V_PAD, _C)[:_V][None]
